# Initial kernel scaffold; baseline (speedup 1.0000x reference)
import jax
import jax.numpy as jnp
from jax.experimental import pallas as pl


def kernel(x, edge_index, W1, b1, W2, b2, W3, b3):
    raise NotImplementedError("write your pallas kernel here")


# speedup vs baseline: 1.9632x; 1.9632x over previous
import functools

import jax
import jax.numpy as jnp
from jax import lax
from jax.experimental import pallas as pl
from jax.experimental.pallas import tpu as pltpu
from jax.experimental.pallas import tpu_sc as plsc

N_NODES = 100000
N_EDGES = 1600000
NP = 100352
DUMMY = N_NODES
CW = 8

NTILES = 32
K = 128
EPT = 50048
NB = EPT // K
EPAD = NTILES * EPT
RPT = NP // 16

BR = 3136
TGRID = NP // BR


def _make_deg_kernel():
  mesh = plsc.VectorSubcoreMesh(core_axis_name="c", subcore_axis_name="s")

  @functools.partial(
      pl.kernel,
      mesh=mesh,
      out_type=jax.ShapeDtypeStruct((2, NP, CW), jnp.float32),
      compiler_params=pltpu.CompilerParams(use_tc_tiling_on_sc=False),
      scratch_types=[
          pltpu.VMEM((K,), jnp.int32),
          pltpu.VMEM((K, CW), jnp.float32),
          pltpu.VMEM_SHARED((NP, CW), jnp.float32),
      ],
  )
  def deg_kernel(dst_hbm, ones_hbm, zeros_hbm, out_hbm, didx, ones_v, acc):
    cid = lax.axis_index("c")
    sid = lax.axis_index("s")
    wid = cid * 16 + sid
    ebase = wid * EPT
    r0 = sid * RPT

    pltpu.sync_copy(ones_hbm, ones_v)
    pltpu.sync_copy(zeros_hbm, acc.at[pl.ds(r0, RPT)])
    plsc.subcore_barrier()

    def body(i, _):
      off = ebase + i * K
      pltpu.sync_copy(dst_hbm.at[pl.ds(off, K)], didx)
      pltpu.sync_copy(ones_v, acc.at[didx], add=True)
      return 0

    lax.fori_loop(0, NB, body, 0)
    plsc.subcore_barrier()
    pltpu.sync_copy(acc.at[pl.ds(r0, RPT)], out_hbm.at[cid, pl.ds(r0, RPT)])

  return deg_kernel


def _make_spmm_kernel(C):
  mesh = plsc.VectorSubcoreMesh(core_axis_name="c", subcore_axis_name="s")

  @functools.partial(
      pl.kernel,
      mesh=mesh,
      out_type=jax.ShapeDtypeStruct((2, C, NP, CW), jnp.float32),
      compiler_params=pltpu.CompilerParams(use_tc_tiling_on_sc=False),
      scratch_types=[
          pltpu.VMEM((K,), jnp.int32),
          pltpu.VMEM((K,), jnp.int32),
          pltpu.VMEM((K,), jnp.int32),
          pltpu.VMEM((K, CW), jnp.float32),
          pltpu.VMEM_SHARED((NP, CW), jnp.float32),
          pltpu.SemaphoreType.DMA,
      ],
  )
  def spmm_kernel(u2d, src_hbm, dst_hbm, zeros_hbm, out_hbm,
                  sidx, didx, gidx, rows, acc, sem):
    cid = lax.axis_index("c")
    sid = lax.axis_index("s")
    wid = cid * 16 + sid
    ebase = wid * EPT
    r0 = sid * RPT

    for c in range(C):
      pltpu.sync_copy(zeros_hbm, acc.at[pl.ds(r0, RPT)])
      plsc.subcore_barrier()

      def body(i, _, c=c):
        off = ebase + i * K
        pltpu.sync_copy(src_hbm.at[pl.ds(off, K)], sidx)
        pltpu.sync_copy(dst_hbm.at[pl.ds(off, K)], didx)
        for j in range(K // 16):
          sl = pl.ds(j * 16, 16)
          gidx[sl] = sidx[sl] * C + c
        pltpu.async_copy(u2d.at[gidx], rows, sem).wait()
        pltpu.sync_copy(rows, acc.at[didx], add=True)
        return 0

      lax.fori_loop(0, NB, body, 0)
      plsc.subcore_barrier()
      pltpu.sync_copy(acc.at[pl.ds(r0, RPT)],
                      out_hbm.at[cid, c, pl.ds(r0, RPT)])
      plsc.subcore_barrier()

  return spmm_kernel


def _dinv_of(d0, d1):
  deg = d0[:, 0:1] + d1[:, 0:1] + 1.0
  return lax.rsqrt(deg)


def _row_spec(cols):
  return pl.BlockSpec((BR, cols), lambda i: (i, 0))


def _full_spec(r, c):
  return pl.BlockSpec((r, c), lambda i: (0, 0))


def _make_first_tc(din, dout):
  def body(x_ref, w_ref, d0_ref, d1_ref, u_ref, h_ref):
    dinv = _dinv_of(d0_ref[...], d1_ref[...])
    h = lax.dot_general(x_ref[...], w_ref[...], (((1,), (0,)), ((), ())),
                        precision=lax.Precision.HIGHEST,
                        preferred_element_type=jnp.float32)
    h_ref[...] = h
    u_ref[...] = h * dinv

  return pl.pallas_call(
      body,
      grid=(TGRID,),
      in_specs=[
          _row_spec(din),
          _full_spec(din, dout),
          _row_spec(CW),
          _row_spec(CW),
      ],
      out_specs=[_row_spec(dout), _row_spec(dout)],
      out_shape=[
          jax.ShapeDtypeStruct((NP, dout), jnp.float32),
          jax.ShapeDtypeStruct((NP, dout), jnp.float32),
      ],
  )


def _combine(accp, C):
  p0 = accp[0].transpose(1, 0, 2).reshape(NP, C * CW)
  p1 = accp[1].transpose(1, 0, 2).reshape(NP, C * CW)
  return p0, p1


def _make_mid_tc(din, dout):
  def body(*refs):
    p0_ref, p1_ref, h_ref, d0_ref, d1_ref, w_ref, b_ref, u_ref, hn_ref = refs
    dinv = _dinv_of(d0_ref[...], d1_ref[...])
    acc = p0_ref[...] + p1_ref[...]
    h = h_ref[...]
    prev = dinv * acc + (dinv * dinv) * h + b_ref[...]
    prev = jnp.maximum(prev, 0.0)
    hn = lax.dot_general(prev, w_ref[...], (((1,), (0,)), ((), ())),
                         precision=lax.Precision.HIGHEST,
                         preferred_element_type=jnp.float32)
    hn_ref[...] = hn
    u_ref[...] = hn * dinv

  return pl.pallas_call(
      body,
      grid=(TGRID,),
      in_specs=(
          [_row_spec(din), _row_spec(din), _row_spec(din),
           _row_spec(CW), _row_spec(CW),
           _full_spec(din, dout), _full_spec(1, din)]
      ),
      out_specs=[_row_spec(dout), _row_spec(dout)],
      out_shape=[
          jax.ShapeDtypeStruct((NP, dout), jnp.float32),
          jax.ShapeDtypeStruct((NP, dout), jnp.float32),
      ],
  )


def _make_last_tc(dout):
  def body(*refs):
    p0_ref, p1_ref, h_ref, d0_ref, d1_ref, b_ref, o_ref = refs
    dinv = _dinv_of(d0_ref[...], d1_ref[...])
    acc = p0_ref[...] + p1_ref[...]
    o_ref[...] = dinv * acc + (dinv * dinv) * h_ref[...] + b_ref[...]

  return pl.pallas_call(
      body,
      grid=(TGRID,),
      in_specs=(
          [_row_spec(dout), _row_spec(dout), _row_spec(dout),
           _row_spec(CW), _row_spec(CW),
           _full_spec(1, dout)]
      ),
      out_specs=_row_spec(dout),
      out_shape=jax.ShapeDtypeStruct((NP, dout), jnp.float32),
  )


@jax.jit
def kernel(x, edge_index, W1, b1, W2, b2, W3, b3):
  src = edge_index[0]
  dst = edge_index[1]
  pad = EPAD - N_EDGES
  src_p = jnp.concatenate([src, jnp.zeros((pad,), jnp.int32)])
  dst_p = jnp.concatenate([dst, jnp.full((pad,), DUMMY, jnp.int32)])
  x_p = jnp.pad(x, ((0, NP - N_NODES), (0, 0)))
  zeros_h = jnp.zeros((RPT, CW), jnp.float32)
  ones_h = jnp.ones((K, CW), jnp.float32)

  degp = _make_deg_kernel()(dst_p, ones_h, zeros_h)
  d0, d1 = degp[0], degp[1]

  u1, h1 = _make_first_tc(8, 64)(x_p, W1, d0, d1)
  acc1 = _make_spmm_kernel(8)(u1.reshape(NP * 8, CW), src_p, dst_p, zeros_h)

  p0, p1 = _combine(acc1, 8)
  u2, h2 = _make_mid_tc(64, 64)(p0, p1, h1, d0, d1, W2, b1.reshape(1, 64))
  acc2 = _make_spmm_kernel(8)(u2.reshape(NP * 8, CW), src_p, dst_p, zeros_h)

  p0, p1 = _combine(acc2, 8)
  u3, h3 = _make_mid_tc(64, 112)(p0, p1, h2, d0, d1, W3, b2.reshape(1, 64))
  acc3 = _make_spmm_kernel(14)(u3.reshape(NP * 14, CW), src_p, dst_p, zeros_h)

  p0, p1 = _combine(acc3, 14)
  out = _make_last_tc(112)(p0, p1, h3, d0, d1, b3.reshape(1, 112))
  return out[:N_NODES]


# speedup vs baseline: 4.4718x; 2.2778x over previous
import functools

import jax
import jax.numpy as jnp
from jax import lax
from jax.experimental import pallas as pl
from jax.experimental.pallas import tpu as pltpu
from jax.experimental.pallas import tpu_sc as plsc

N_NODES = 100000
N_EDGES = 1600000
NP = 100352
DUMMY = N_NODES
CW = 8

NTILES = 32
K = 128
EPT = 50176
NB = EPT // K
EPAD = NTILES * EPT
RPT = NP // 16

BR = 3136
TGRID = NP // BR


def _make_deg_kernel():
  mesh = plsc.VectorSubcoreMesh(core_axis_name="c", subcore_axis_name="s")

  @functools.partial(
      pl.kernel,
      mesh=mesh,
      out_type=jax.ShapeDtypeStruct((2, NP, CW), jnp.float32),
      compiler_params=pltpu.CompilerParams(use_tc_tiling_on_sc=False),
      scratch_types=[
          pltpu.VMEM((NB, K), jnp.int32),
          pltpu.VMEM((K, CW), jnp.float32),
          pltpu.VMEM_SHARED((NP, CW), jnp.float32),
      ],
  )
  def deg_kernel(dst_hbm, ones_hbm, zeros_hbm, out_hbm, didx2d, ones_v, acc):
    cid = lax.axis_index("c")
    sid = lax.axis_index("s")
    wid = cid * 16 + sid
    r0 = sid * RPT

    pltpu.sync_copy(ones_hbm, ones_v)
    pltpu.sync_copy(dst_hbm.at[pl.ds(wid * NB, NB)], didx2d)
    pltpu.sync_copy(zeros_hbm, acc.at[pl.ds(r0, RPT)])
    plsc.subcore_barrier()

    def body(i, _):
      pltpu.sync_copy(ones_v, acc.at[didx2d.at[i]], add=True)
      return 0

    lax.fori_loop(0, NB, body, 0)
    plsc.subcore_barrier()
    pltpu.sync_copy(acc.at[pl.ds(r0, RPT)], out_hbm.at[cid, pl.ds(r0, RPT)])

  return deg_kernel


def _make_spmm_kernel(C):
  mesh = plsc.VectorSubcoreMesh(core_axis_name="c", subcore_axis_name="s")

  @functools.partial(
      pl.kernel,
      mesh=mesh,
      out_type=jax.ShapeDtypeStruct((2, C, NP, CW), jnp.float32),
      compiler_params=pltpu.CompilerParams(use_tc_tiling_on_sc=False),
      scratch_types=[
          pltpu.VMEM((NB, K), jnp.int32),
          pltpu.VMEM((2, K), jnp.int32),
          pltpu.VMEM((2, K), jnp.int32),
          pltpu.VMEM((2, K, CW), jnp.float32),
          pltpu.VMEM_SHARED((NP, CW), jnp.float32),
          pltpu.SemaphoreType.DMA,
          pltpu.SemaphoreType.DMA,
          pltpu.SemaphoreType.DMA,
          pltpu.SemaphoreType.DMA,
      ],
  )
  def spmm_kernel(u2d, src_hbm, dst_hbm, zeros_hbm, out_hbm,
                  didx2d, sidx, gidx, rows, acc, gs0, gs1, ss0, ss1):
    cid = lax.axis_index("c")
    sid = lax.axis_index("s")
    wid = cid * 16 + sid
    bbase = wid * NB
    r0 = sid * RPT
    gsem = (gs0, gs1)
    ssem = (ss0, ss1)

    pltpu.sync_copy(dst_hbm.at[pl.ds(bbase, NB)], didx2d)

    def fire_sidx(i, p):
      pltpu.async_copy(src_hbm.at[bbase + i], sidx.at[p], ssem[p])

    def wait_sidx(i, p):
      pltpu.make_async_copy(src_hbm.at[bbase + i], sidx.at[p],
                            ssem[p]).wait()

    def fire_gather(p, c):
      for j in range(K // 16):
        sl = pl.ds(j * 16, 16)
        gidx[p, sl] = sidx[p, sl] * C + c
      pltpu.async_copy(u2d.at[gidx.at[p]], rows.at[p], gsem[p])

    def wait_gather(p):
      pltpu.make_async_copy(u2d.at[gidx.at[p]], rows.at[p], gsem[p]).wait()

    def scatter(i, p):
      pltpu.sync_copy(rows.at[p], acc.at[didx2d.at[i]], add=True)

    for c in range(C):
      pltpu.sync_copy(zeros_hbm, acc.at[pl.ds(r0, RPT)])
      plsc.subcore_barrier()

      fire_sidx(0, 0)
      fire_sidx(1, 1)
      wait_sidx(0, 0)
      fire_gather(0, c)

      def pair(t, _, c=c):
        i0 = 2 * t
        fire_sidx(i0 + 2, 0)
        wait_sidx(i0 + 1, 1)
        fire_gather(1, c)
        wait_gather(0)
        scatter(i0, 0)
        fire_sidx(i0 + 3, 1)
        wait_sidx(i0 + 2, 0)
        fire_gather(0, c)
        wait_gather(1)
        scatter(i0 + 1, 1)
        return 0

      lax.fori_loop(0, NB // 2, pair, 0)
      wait_sidx(NB + 1, 1)
      wait_gather(0)
      plsc.subcore_barrier()
      pltpu.sync_copy(acc.at[pl.ds(r0, RPT)],
                      out_hbm.at[cid, c, pl.ds(r0, RPT)])
      plsc.subcore_barrier()

  return spmm_kernel


def _dinv_of(d0, d1):
  deg = d0[:, 0:1] + d1[:, 0:1] + 1.0
  return lax.rsqrt(deg)


def _row_spec(cols):
  return pl.BlockSpec((BR, cols), lambda i: (i, 0))


def _full_spec(r, c):
  return pl.BlockSpec((r, c), lambda i: (0, 0))


def _make_first_tc(din, dout):
  def body(x_ref, w_ref, d0_ref, d1_ref, u_ref, h_ref):
    dinv = _dinv_of(d0_ref[...], d1_ref[...])
    h = lax.dot_general(x_ref[...], w_ref[...], (((1,), (0,)), ((), ())),
                        precision=lax.Precision.HIGHEST,
                        preferred_element_type=jnp.float32)
    h_ref[...] = h
    u_ref[...] = h * dinv

  return pl.pallas_call(
      body,
      grid=(TGRID,),
      in_specs=[
          _row_spec(din),
          _full_spec(din, dout),
          _row_spec(CW),
          _row_spec(CW),
      ],
      out_specs=[_row_spec(dout), _row_spec(dout)],
      out_shape=[
          jax.ShapeDtypeStruct((NP, dout), jnp.float32),
          jax.ShapeDtypeStruct((NP, dout), jnp.float32),
      ],
  )


def _combine(accp, C):
  p0 = accp[0].transpose(1, 0, 2).reshape(NP, C * CW)
  p1 = accp[1].transpose(1, 0, 2).reshape(NP, C * CW)
  return p0, p1


def _make_mid_tc(din, dout):
  def body(*refs):
    p0_ref, p1_ref, h_ref, d0_ref, d1_ref, w_ref, b_ref, u_ref, hn_ref = refs
    dinv = _dinv_of(d0_ref[...], d1_ref[...])
    acc = p0_ref[...] + p1_ref[...]
    h = h_ref[...]
    prev = dinv * acc + (dinv * dinv) * h + b_ref[...]
    prev = jnp.maximum(prev, 0.0)
    hn = lax.dot_general(prev, w_ref[...], (((1,), (0,)), ((), ())),
                         precision=lax.Precision.HIGHEST,
                         preferred_element_type=jnp.float32)
    hn_ref[...] = hn
    u_ref[...] = hn * dinv

  return pl.pallas_call(
      body,
      grid=(TGRID,),
      in_specs=(
          [_row_spec(din), _row_spec(din), _row_spec(din),
           _row_spec(CW), _row_spec(CW),
           _full_spec(din, dout), _full_spec(1, din)]
      ),
      out_specs=[_row_spec(dout), _row_spec(dout)],
      out_shape=[
          jax.ShapeDtypeStruct((NP, dout), jnp.float32),
          jax.ShapeDtypeStruct((NP, dout), jnp.float32),
      ],
  )


def _make_last_tc(dout):
  def body(*refs):
    p0_ref, p1_ref, h_ref, d0_ref, d1_ref, b_ref, o_ref = refs
    dinv = _dinv_of(d0_ref[...], d1_ref[...])
    acc = p0_ref[...] + p1_ref[...]
    o_ref[...] = dinv * acc + (dinv * dinv) * h_ref[...] + b_ref[...]

  return pl.pallas_call(
      body,
      grid=(TGRID,),
      in_specs=(
          [_row_spec(dout), _row_spec(dout), _row_spec(dout),
           _row_spec(CW), _row_spec(CW),
           _full_spec(1, dout)]
      ),
      out_specs=_row_spec(dout),
      out_shape=jax.ShapeDtypeStruct((NP, dout), jnp.float32),
  )


@jax.jit
def kernel(x, edge_index, W1, b1, W2, b2, W3, b3):
  src = edge_index[0]
  dst = edge_index[1]
  pad = EPAD - N_EDGES
  src_p = jnp.concatenate(
      [src, jnp.zeros((pad + 2 * K,), jnp.int32)]).reshape(-1, K)
  dst_p = jnp.concatenate(
      [dst, jnp.full((pad,), DUMMY, jnp.int32)]).reshape(-1, K)
  x_p = jnp.pad(x, ((0, NP - N_NODES), (0, 0)))
  zeros_h = jnp.zeros((RPT, CW), jnp.float32)
  ones_h = jnp.ones((K, CW), jnp.float32)

  degp = _make_deg_kernel()(dst_p, ones_h, zeros_h)
  d0, d1 = degp[0], degp[1]

  u1, h1 = _make_first_tc(8, 64)(x_p, W1, d0, d1)
  acc1 = _make_spmm_kernel(8)(u1.reshape(NP * 8, CW), src_p, dst_p, zeros_h)

  p0, p1 = _combine(acc1, 8)
  u2, h2 = _make_mid_tc(64, 64)(p0, p1, h1, d0, d1, W2, b1.reshape(1, 64))
  acc2 = _make_spmm_kernel(8)(u2.reshape(NP * 8, CW), src_p, dst_p, zeros_h)

  p0, p1 = _combine(acc2, 8)
  u3, h3 = _make_mid_tc(64, 112)(p0, p1, h2, d0, d1, W3, b2.reshape(1, 64))
  acc3 = _make_spmm_kernel(14)(u3.reshape(NP * 14, CW), src_p, dst_p, zeros_h)

  p0, p1 = _combine(acc3, 14)
  out = _make_last_tc(112)(p0, p1, h3, d0, d1, b3.reshape(1, 112))
  return out[:N_NODES]


# speedup vs baseline: 4.6170x; 1.0325x over previous
import functools

import jax
import jax.numpy as jnp
from jax import lax
from jax.experimental import pallas as pl
from jax.experimental.pallas import tpu as pltpu
from jax.experimental.pallas import tpu_sc as plsc

N_NODES = 100000
N_EDGES = 1600000
NP = 100352
DUMMY = N_NODES
CW = 8

NTILES = 32
K = 128
EPT = 50176
NB = EPT // K
EPAD = NTILES * EPT
RPT = NP // 16

BR = 3136
TGRID = NP // BR


def _make_deg_kernel():
  mesh = plsc.VectorSubcoreMesh(core_axis_name="c", subcore_axis_name="s")

  @functools.partial(
      pl.kernel,
      mesh=mesh,
      out_type=jax.ShapeDtypeStruct((2, NP, CW), jnp.float32),
      compiler_params=pltpu.CompilerParams(use_tc_tiling_on_sc=False),
      scratch_types=[
          pltpu.VMEM((NB, K), jnp.int32),
          pltpu.VMEM((K, CW), jnp.float32),
          pltpu.VMEM_SHARED((NP, CW), jnp.float32),
      ],
  )
  def deg_kernel(dst_hbm, ones_hbm, zeros_hbm, out_hbm, didx2d, ones_v, acc):
    cid = lax.axis_index("c")
    sid = lax.axis_index("s")
    wid = cid * 16 + sid
    r0 = sid * RPT

    pltpu.sync_copy(ones_hbm, ones_v)
    pltpu.sync_copy(dst_hbm.at[pl.ds(wid * NB, NB)], didx2d)
    pltpu.sync_copy(zeros_hbm, acc.at[pl.ds(r0, RPT)])
    plsc.subcore_barrier()

    def body(i, _):
      pltpu.sync_copy(ones_v, acc.at[didx2d.at[i]], add=True)
      return 0

    lax.fori_loop(0, NB, body, 0)
    plsc.subcore_barrier()
    pltpu.sync_copy(acc.at[pl.ds(r0, RPT)], out_hbm.at[cid, pl.ds(r0, RPT)])

  return deg_kernel


def _make_spmm_kernel(C):
  mesh = plsc.VectorSubcoreMesh(core_axis_name="c", subcore_axis_name="s")

  @functools.partial(
      pl.kernel,
      mesh=mesh,
      out_type=jax.ShapeDtypeStruct((2, C, NP, CW), jnp.float32),
      compiler_params=pltpu.CompilerParams(use_tc_tiling_on_sc=False),
      scratch_types=[
          pltpu.VMEM((NB, K), jnp.int32),
          pltpu.VMEM((4, K), jnp.int32),
          pltpu.VMEM((4, K), jnp.int32),
          pltpu.VMEM((4, K, CW), jnp.float32),
          pltpu.VMEM_SHARED((NP, CW), jnp.float32),
      ] + [pltpu.SemaphoreType.DMA] * 12,
  )
  def spmm_kernel(u2d, src_hbm, dst_hbm, zeros_hbm, out_hbm,
                  didx2d, sidx, gidx, rows, acc, *sems):
    cid = lax.axis_index("c")
    sid = lax.axis_index("s")
    wid = cid * 16 + sid
    bbase = wid * NB
    r0 = sid * RPT
    gsem = sems[0:4]
    ssem = sems[4:8]
    wsem = sems[8:12]

    pltpu.sync_copy(dst_hbm.at[pl.ds(bbase, NB)], didx2d)

    def fire_sidx(i, b):
      pltpu.async_copy(src_hbm.at[bbase + i], sidx.at[b], ssem[b])

    def wait_sidx(i, b):
      pltpu.make_async_copy(src_hbm.at[bbase + i], sidx.at[b],
                            ssem[b]).wait()

    def fire_gather(b, c):
      for j in range(K // 16):
        sl = pl.ds(j * 16, 16)
        gidx[b, sl] = sidx[b, sl] * C + c
      pltpu.async_copy(u2d.at[gidx.at[b]], rows.at[b], gsem[b])

    def wait_gather(b):
      pltpu.make_async_copy(u2d.at[gidx.at[b]], rows.at[b], gsem[b]).wait()

    def fire_scatter(i, b):
      pltpu.async_copy(rows.at[b], acc.at[didx2d.at[i]], wsem[b], add=True)

    def wait_scatter(b):
      pltpu.make_async_copy(rows.at[b], acc.at[didx2d.at[0]],
                            wsem[b]).wait()

    for c in range(C):
      pltpu.sync_copy(zeros_hbm, acc.at[pl.ds(r0, RPT)])
      plsc.subcore_barrier()

      fire_sidx(0, 0)
      fire_sidx(1, 1)
      wait_sidx(0, 0)
      fire_gather(0, c)

      def quad(t, _, c=c):
        for ph in range(4):
          i = 4 * t + ph
          bn = (ph + 1) % 4
          fire_sidx(i + 2, (ph + 2) % 4)
          wait_sidx(i + 1, bn)
          if ph == 3:
            wait_scatter(bn)
          else:
            @pl.when(t > 0)
            def _():
              wait_scatter(bn)
          fire_gather(bn, c)
          wait_gather(ph)
          fire_scatter(i, ph)
        return 0

      lax.fori_loop(0, NB // 4, quad, 0)
      wait_scatter(1)
      wait_scatter(2)
      wait_scatter(3)
      wait_gather(0)
      wait_sidx(NB + 1, 1)
      plsc.subcore_barrier()
      pltpu.sync_copy(acc.at[pl.ds(r0, RPT)],
                      out_hbm.at[cid, c, pl.ds(r0, RPT)])
      plsc.subcore_barrier()

  return spmm_kernel


def _dinv_of(d0, d1):
  deg = d0[:, 0:1] + d1[:, 0:1] + 1.0
  return lax.rsqrt(deg)


def _row_spec(cols):
  return pl.BlockSpec((BR, cols), lambda i: (i, 0))


def _full_spec(r, c):
  return pl.BlockSpec((r, c), lambda i: (0, 0))


def _make_first_tc(din, dout):
  def body(x_ref, w_ref, d0_ref, d1_ref, u_ref, h_ref):
    dinv = _dinv_of(d0_ref[...], d1_ref[...])
    h = lax.dot_general(x_ref[...], w_ref[...], (((1,), (0,)), ((), ())),
                        precision=lax.Precision.HIGHEST,
                        preferred_element_type=jnp.float32)
    h_ref[...] = h
    u_ref[...] = h * dinv

  return pl.pallas_call(
      body,
      grid=(TGRID,),
      in_specs=[
          _row_spec(din),
          _full_spec(din, dout),
          _row_spec(CW),
          _row_spec(CW),
      ],
      out_specs=[_row_spec(dout), _row_spec(dout)],
      out_shape=[
          jax.ShapeDtypeStruct((NP, dout), jnp.float32),
          jax.ShapeDtypeStruct((NP, dout), jnp.float32),
      ],
  )


def _combine(accp, C):
  p0 = accp[0].transpose(1, 0, 2).reshape(NP, C * CW)
  p1 = accp[1].transpose(1, 0, 2).reshape(NP, C * CW)
  return p0, p1


def _make_mid_tc(din, dout):
  def body(*refs):
    p0_ref, p1_ref, h_ref, d0_ref, d1_ref, w_ref, b_ref, u_ref, hn_ref = refs
    dinv = _dinv_of(d0_ref[...], d1_ref[...])
    acc = p0_ref[...] + p1_ref[...]
    h = h_ref[...]
    prev = dinv * acc + (dinv * dinv) * h + b_ref[...]
    prev = jnp.maximum(prev, 0.0)
    hn = lax.dot_general(prev, w_ref[...], (((1,), (0,)), ((), ())),
                         precision=lax.Precision.HIGHEST,
                         preferred_element_type=jnp.float32)
    hn_ref[...] = hn
    u_ref[...] = hn * dinv

  return pl.pallas_call(
      body,
      grid=(TGRID,),
      in_specs=(
          [_row_spec(din), _row_spec(din), _row_spec(din),
           _row_spec(CW), _row_spec(CW),
           _full_spec(din, dout), _full_spec(1, din)]
      ),
      out_specs=[_row_spec(dout), _row_spec(dout)],
      out_shape=[
          jax.ShapeDtypeStruct((NP, dout), jnp.float32),
          jax.ShapeDtypeStruct((NP, dout), jnp.float32),
      ],
  )


def _make_last_tc(dout):
  def body(*refs):
    p0_ref, p1_ref, h_ref, d0_ref, d1_ref, b_ref, o_ref = refs
    dinv = _dinv_of(d0_ref[...], d1_ref[...])
    acc = p0_ref[...] + p1_ref[...]
    o_ref[...] = dinv * acc + (dinv * dinv) * h_ref[...] + b_ref[...]

  return pl.pallas_call(
      body,
      grid=(TGRID,),
      in_specs=(
          [_row_spec(dout), _row_spec(dout), _row_spec(dout),
           _row_spec(CW), _row_spec(CW),
           _full_spec(1, dout)]
      ),
      out_specs=_row_spec(dout),
      out_shape=jax.ShapeDtypeStruct((NP, dout), jnp.float32),
  )


@jax.jit
def kernel(x, edge_index, W1, b1, W2, b2, W3, b3):
  src = edge_index[0]
  dst = edge_index[1]
  pad = EPAD - N_EDGES
  src_p = jnp.concatenate(
      [src, jnp.zeros((pad + 2 * K,), jnp.int32)]).reshape(-1, K)
  dst_p = jnp.concatenate(
      [dst, jnp.full((pad,), DUMMY, jnp.int32)]).reshape(-1, K)
  x_p = jnp.pad(x, ((0, NP - N_NODES), (0, 0)))
  zeros_h = jnp.zeros((RPT, CW), jnp.float32)
  ones_h = jnp.ones((K, CW), jnp.float32)

  degp = _make_deg_kernel()(dst_p, ones_h, zeros_h)
  d0, d1 = degp[0], degp[1]

  u1, h1 = _make_first_tc(8, 64)(x_p, W1, d0, d1)
  acc1 = _make_spmm_kernel(8)(u1.reshape(NP * 8, CW), src_p, dst_p, zeros_h)

  p0, p1 = _combine(acc1, 8)
  u2, h2 = _make_mid_tc(64, 64)(p0, p1, h1, d0, d1, W2, b1.reshape(1, 64))
  acc2 = _make_spmm_kernel(8)(u2.reshape(NP * 8, CW), src_p, dst_p, zeros_h)

  p0, p1 = _combine(acc2, 8)
  u3, h3 = _make_mid_tc(64, 112)(p0, p1, h2, d0, d1, W3, b2.reshape(1, 64))
  acc3 = _make_spmm_kernel(14)(u3.reshape(NP * 14, CW), src_p, dst_p, zeros_h)

  p0, p1 = _combine(acc3, 14)
  out = _make_last_tc(112)(p0, p1, h3, d0, d1, b3.reshape(1, 112))
  return out[:N_NODES]
